# baseline (speedup 1.0000x reference)
import jax
import jax.numpy as jnp
from jax.experimental import pallas as pl
from jax.experimental.pallas import tpu as pltpu


def _round_up(x, m):
    return ((x + m - 1) // m) * m


_NSLOTS = 4
_AHEAD = 3


def _make_embed_kernel(tile_tokens, n_pos_tiles, n_tiles):
    T = tile_tokens

    def _issue(ids_ref, wte_hbm, gbuf, sems, tile_idx, slot):
        base = tile_idx * T
        for i in range(T):
            tok = ids_ref[base + i]
            pltpu.make_async_copy(
                wte_hbm.at[pl.ds(tok, 1), :],
                gbuf.at[slot, pl.ds(i, 1), :],
                sems.at[slot],
            ).start()

    def _kernel(ids_ref, wpe_ref, wte_hbm, out_ref, gbuf, sems):
        j = pl.program_id(0)
        slot = j % _NSLOTS

        @pl.when(j == 0)
        def _():
            _issue(ids_ref, wte_hbm, gbuf, sems, j, 0)

        for k in range(1, _AHEAD):
            if n_tiles > k:
                @pl.when(j == 0)
                def _(_k=k):
                    _issue(ids_ref, wte_hbm, gbuf, sems, j + _k, _k % _NSLOTS)

        @pl.when(j + _AHEAD < n_tiles)
        def _():
            _issue(
                ids_ref, wte_hbm, gbuf, sems,
                j + _AHEAD, (j + _AHEAD) % _NSLOTS,
            )

        pltpu.make_async_copy(
            gbuf.at[slot], gbuf.at[slot], sems.at[slot]
        ).wait()

        pos = pl.multiple_of((j % n_pos_tiles) * T, 8)
        out_ref[...] = gbuf[slot] + wpe_ref[pl.ds(pos, T), :]

    return _kernel


def _embed_gather(input_ids, wte, wpe, *, max_tile_tokens=256):
    B, S = input_ids.shape
    V, E = wte.shape
    dtype = wte.dtype

    T = max(8, (min(max_tile_tokens, _round_up(S, 8)) // 8) * 8)
    S_pad = _round_up(S, T)
    n_pos_tiles = S_pad // T
    n_tiles = (B * S_pad) // T

    ids = input_ids.astype(jnp.int32)
    if S_pad != S:
        ids = jnp.pad(ids, ((0, 0), (0, S_pad - S)))
    ids_flat = ids.reshape(B * S_pad)

    if wpe.shape[0] == S_pad and wpe.dtype == dtype:
        wpe_s = wpe
    else:
        wpe_s = wpe[:S].astype(dtype)
        if S_pad != S:
            wpe_s = jnp.pad(wpe_s, ((0, S_pad - S), (0, 0)))

    kernel_fn = _make_embed_kernel(T, n_pos_tiles, n_tiles)

    grid_spec = pltpu.PrefetchScalarGridSpec(
        num_scalar_prefetch=1,
        grid=(n_tiles,),
        in_specs=[
            pl.BlockSpec((S_pad, E), lambda j, ids: (0, 0)),
            pl.BlockSpec(memory_space=pl.ANY),
        ],
        out_specs=pl.BlockSpec((T, E), lambda j, ids: (j, 0)),
        scratch_shapes=[
            pltpu.VMEM((_NSLOTS, T, E), dtype),
            pltpu.SemaphoreType.DMA((_NSLOTS,)),
        ],
    )

    out_flat = pl.pallas_call(
        kernel_fn,
        out_shape=jax.ShapeDtypeStruct((B * S_pad, E), dtype),
        grid_spec=grid_spec,
        compiler_params=pltpu.CompilerParams(
            dimension_semantics=("arbitrary",),
            vmem_limit_bytes=48 * 1024 * 1024,
            disable_bounds_checks=True,
        ),
    )(ids_flat, wpe_s, wte)

    out = out_flat.reshape(B, S_pad, E)
    if S_pad != S:
        out = out[:, :S, :]
    return out


def kernel(input_ids, wte, wpe):
    return _embed_gather(input_ids, wte, wpe)


# speedup vs baseline: 1.0089x; 1.0089x over previous
import jax
import jax.numpy as jnp
from jax.experimental import pallas as pl
from jax.experimental.pallas import tpu as pltpu


def _round_up(x, m):
    return ((x + m - 1) // m) * m


_NSLOTS = 4
_AHEAD = 3


def _make_embed_kernel(tile_tokens, n_pos_tiles, n_tiles):
    T = tile_tokens

    def _issue(ids_ref, wte_hbm, gbuf, sems, tile_idx, slot):
        base = tile_idx * T
        for i in range(T):
            tok = ids_ref[base + i]
            pltpu.make_async_copy(
                wte_hbm.at[pl.ds(tok, 1), :],
                gbuf.at[slot, pl.ds(i, 1), :],
                sems.at[slot],
            ).start()

    def _kernel(ids_ref, wpe_hbm, wte_hbm, out_ref, gbuf, wpe_buf, sems,
                wpe_sem):
        j = pl.program_id(0)
        slot = j % _NSLOTS

        @pl.when(j == 0)
        def _():
            _issue(ids_ref, wte_hbm, gbuf, sems, j, 0)
            pltpu.make_async_copy(wpe_hbm, wpe_buf, wpe_sem).start()

        for k in range(1, _AHEAD):
            if n_tiles > k:
                @pl.when(j == 0)
                def _(_k=k):
                    _issue(ids_ref, wte_hbm, gbuf, sems, j + _k, _k % _NSLOTS)

        @pl.when(j + _AHEAD < n_tiles)
        def _():
            _issue(
                ids_ref, wte_hbm, gbuf, sems,
                j + _AHEAD, (j + _AHEAD) % _NSLOTS,
            )

        @pl.when(j == 0)
        def _():
            pltpu.make_async_copy(wpe_hbm, wpe_buf, wpe_sem).wait()

        pltpu.make_async_copy(
            gbuf.at[slot], gbuf.at[slot], sems.at[slot]
        ).wait()

        pos = pl.multiple_of((j % n_pos_tiles) * T, 8)
        out_ref[...] = gbuf[slot] + wpe_buf[pl.ds(pos, T), :]

    return _kernel


def _embed_gather(input_ids, wte, wpe, *, max_tile_tokens=512):
    B, S = input_ids.shape
    V, E = wte.shape
    dtype = wte.dtype

    T = max(8, (min(max_tile_tokens, _round_up(S, 8)) // 8) * 8)
    S_pad = _round_up(S, T)
    n_pos_tiles = S_pad // T
    n_tiles = (B * S_pad) // T

    ids = input_ids.astype(jnp.int32)
    if S_pad != S:
        ids = jnp.pad(ids, ((0, 0), (0, S_pad - S)))
    ids_flat = ids.reshape(B * S_pad)

    if wpe.shape[0] == S_pad and wpe.dtype == dtype:
        wpe_s = wpe
    else:
        wpe_s = wpe[:S].astype(dtype)
        if S_pad != S:
            wpe_s = jnp.pad(wpe_s, ((0, S_pad - S), (0, 0)))

    kernel_fn = _make_embed_kernel(T, n_pos_tiles, n_tiles)

    grid_spec = pltpu.PrefetchScalarGridSpec(
        num_scalar_prefetch=1,
        grid=(n_tiles,),
        in_specs=[
            pl.BlockSpec(memory_space=pl.ANY),
            pl.BlockSpec(memory_space=pl.ANY),
        ],
        out_specs=pl.BlockSpec((T, E), lambda j, ids: (j, 0)),
        scratch_shapes=[
            pltpu.VMEM((_NSLOTS, T, E), dtype),
            pltpu.VMEM((S_pad, E), dtype),
            pltpu.SemaphoreType.DMA((_NSLOTS,)),
            pltpu.SemaphoreType.DMA,
        ],
    )

    out_flat = pl.pallas_call(
        kernel_fn,
        out_shape=jax.ShapeDtypeStruct((B * S_pad, E), dtype),
        grid_spec=grid_spec,
        compiler_params=pltpu.CompilerParams(
            dimension_semantics=("arbitrary",),
            vmem_limit_bytes=48 * 1024 * 1024,
            disable_bounds_checks=True,
        ),
    )(ids_flat, wpe_s, wte)

    out = out_flat.reshape(B, S_pad, E)
    if S_pad != S:
        out = out[:, :S, :]
    return out


def kernel(input_ids, wte, wpe):
    return _embed_gather(input_ids, wte, wpe)


# speedup vs baseline: 1.0198x; 1.0109x over previous
import jax
import jax.numpy as jnp
from jax.experimental import pallas as pl
from jax.experimental.pallas import tpu as pltpu


def _round_up(x, m):
    return ((x + m - 1) // m) * m


_NSLOTS = 4
_AHEAD = 3


def _make_embed_kernel(tile_tokens, n_pos_tiles, n_tiles):
    T = tile_tokens

    def _issue(ids_ref, wte_hbm, gbuf, sems, tile_idx, slot):
        base = tile_idx * T
        for i in range(T):
            tok = ids_ref[base + i]
            pltpu.make_async_copy(
                wte_hbm.at[pl.ds(tok, 1), :],
                gbuf.at[slot, pl.ds(i, 1), :],
                sems.at[slot],
            ).start()

    def _kernel(ids_ref, wpe_hbm, wte_hbm, out_ref, gbuf, wpe_buf, sems,
                wpe_sem):
        j = pl.program_id(0)
        slot = j % _NSLOTS

        @pl.when(j == 0)
        def _():
            _issue(ids_ref, wte_hbm, gbuf, sems, j, 0)
            pltpu.make_async_copy(wpe_hbm, wpe_buf, wpe_sem).start()

        for k in range(1, _AHEAD):
            if n_tiles > k:
                @pl.when(j == 0)
                def _(_k=k):
                    _issue(ids_ref, wte_hbm, gbuf, sems, j + _k, _k % _NSLOTS)

        @pl.when(j + _AHEAD < n_tiles)
        def _():
            _issue(
                ids_ref, wte_hbm, gbuf, sems,
                j + _AHEAD, (j + _AHEAD) % _NSLOTS,
            )

        @pl.when(j == 0)
        def _():
            pltpu.make_async_copy(wpe_hbm, wpe_buf, wpe_sem).wait()

        pltpu.make_async_copy(
            gbuf.at[slot], gbuf.at[slot], sems.at[slot]
        ).wait()

        pos = pl.multiple_of((j % n_pos_tiles) * T, 8)
        out_ref[...] = gbuf[slot] + wpe_buf[pl.ds(pos, T), :]

    return _kernel


def _embed_gather(input_ids, wte, wpe, *, max_tile_tokens=512):
    B, S = input_ids.shape
    V, E = wte.shape
    dtype = wte.dtype

    T = max(8, (min(max_tile_tokens, _round_up(S, 8)) // 8) * 8)
    S_pad = _round_up(S, T)
    n_pos_tiles = S_pad // T
    n_tiles = (B * S_pad) // T

    ids = input_ids.astype(jnp.int32)
    if S_pad != S:
        ids = jnp.pad(ids, ((0, 0), (0, S_pad - S)))
    ids_flat = ids.reshape(B * S_pad)

    if wpe.shape[0] == S_pad and wpe.dtype == dtype:
        wpe_s = wpe
    else:
        wpe_s = wpe[:S].astype(dtype)
        if S_pad != S:
            wpe_s = jnp.pad(wpe_s, ((0, S_pad - S), (0, 0)))

    kernel_fn = _make_embed_kernel(T, n_pos_tiles, n_tiles)

    grid_spec = pltpu.PrefetchScalarGridSpec(
        num_scalar_prefetch=1,
        grid=(n_tiles,),
        in_specs=[
            pl.BlockSpec(memory_space=pl.ANY),
            pl.BlockSpec(memory_space=pl.ANY),
        ],
        out_specs=pl.BlockSpec((T, E), lambda j, ids: (j, 0)),
        scratch_shapes=[
            pltpu.VMEM((_NSLOTS, T, E), dtype),
            pltpu.VMEM((S_pad, E), dtype),
            pltpu.SemaphoreType.DMA((_NSLOTS,)),
            pltpu.SemaphoreType.DMA,
        ],
    )

    out_flat = pl.pallas_call(
        kernel_fn,
        out_shape=jax.ShapeDtypeStruct((B * S_pad, E), dtype),
        grid_spec=grid_spec,
        compiler_params=pltpu.CompilerParams(
            dimension_semantics=("arbitrary",),
            vmem_limit_bytes=62 * 1024 * 1024,
            disable_bounds_checks=True,
        ),
    )(ids_flat, wpe_s, wte)

    out = out_flat.reshape(B, S_pad, E)
    if S_pad != S:
        out = out[:, :S, :]
    return out


def kernel(input_ids, wte, wpe):
    return _embed_gather(input_ids, wte, wpe)
